# baseline (speedup 1.0000x reference)
import functools

import jax
import jax.numpy as jnp
from jax.experimental import pallas as pl
from jax.experimental.pallas import tpu as pltpu

D_MODEL = 1024
D_SPACE = 64
TOKENS_BLK = 256
OUT_W = 128


def _h_kernel(x_ref, w_ref, b_ref, h_ref):
    h_ref[...] = (
        jnp.dot(x_ref[...], w_ref[...], preferred_element_type=jnp.float32)
        + b_ref[...]
    )


def _pool_kernel(h_ref, embt_ref, w_out_ref, i_out_ref, cur_ref, cv_ref,
                 ci_ref, *, k: int, m: int):
    e = embt_ref[...]
    nrm = jnp.sqrt(jnp.sum(e * e, axis=0, keepdims=True))
    en = e / jnp.maximum(nrm, 1e-12)
    h = h_ref[...]
    logits = jnp.dot(h, en, preferred_element_type=jnp.float32)
    cur_ref[...] = logits
    T, N = logits.shape
    C = N // 128
    lane = jax.lax.broadcasted_iota(jnp.int32, (T, 128), 1)
    outlane = jax.lax.broadcasted_iota(jnp.int32, (T, OUT_W), 1)
    acc_v0 = jnp.full((T, OUT_W), -jnp.inf, dtype=jnp.float32)
    acc_i0 = jnp.zeros((T, OUT_W), dtype=jnp.int32)

    for it in range(m):
        mx = cur_ref[:, 0:128]
        for c in range(1, C):
            mx = jnp.maximum(mx, cur_ref[:, c * 128:(c + 1) * 128])
        ci = jnp.zeros((T, 128), jnp.int32)
        done = jnp.zeros((T, 128), jnp.bool_)
        for c in range(C):
            s = cur_ref[:, c * 128:(c + 1) * 128]
            kill = (s == mx) & (~done)
            ci = jnp.where(kill, c, ci)
            done = done | kill
            if it < m - 1:
                cur_ref[:, c * 128:(c + 1) * 128] = jnp.where(
                    kill, -jnp.inf, s)
        cv_ref[:, it * 128:(it + 1) * 128] = mx
        ci_ref[:, it * 128:(it + 1) * 128] = ci * 128 + lane

    vlast = cv_ref[:, (m - 1) * 128:m * 128]

    BIG = jnp.int32(1 << 30)

    def body(j, carry):
        acc_v, acc_i, hv, hoi, depth, _ = carry
        g = jnp.max(hv, axis=1, keepdims=True)
        eq = hv == g
        win = jnp.min(jnp.where(eq, hoi, BIG), axis=1, keepdims=True)
        winm = eq & (hoi == win)
        nd = depth + winm.astype(jnp.int32)
        t0 = (nd & 1) != 0
        t1 = (nd & 2) != 0
        t2 = (nd & 4) != 0

        def tree(ref):
            cs = [ref[:, c * 128:(c + 1) * 128] for c in range(m)]
            cs = cs + [cs[0]] * (8 - m)
            a = jnp.where(t0, cs[1], cs[0])
            b = jnp.where(t0, cs[3], cs[2])
            c_ = jnp.where(t0, cs[5], cs[4])
            d = jnp.where(t0, cs[7], cs[6])
            e_ = jnp.where(t1, b, a)
            f = jnp.where(t1, d, c_)
            return jnp.where(t2, f, e_)

        newv = jnp.where(nd >= m, -jnp.inf, tree(cv_ref))
        newi = tree(ci_ref)
        hv = jnp.where(winm, newv, hv)
        hoi = jnp.where(winm, newi, hoi)
        sel = outlane == j
        acc_v = jnp.where(sel, g, acc_v)
        acc_i = jnp.where(sel, win, acc_i)
        return acc_v, acc_i, hv, hoi, nd, g

    g0 = jnp.zeros((T, 1), jnp.float32)
    hv0 = cv_ref[:, 0:128]
    hoi0 = ci_ref[:, 0:128]
    d0 = jnp.zeros((T, 128), jnp.int32)
    acc_v, acc_i, _, _, _, gk = jax.lax.fori_loop(
        0, k, body, (acc_v0, acc_i0, hv0, hoi0, d0, g0))

    def finalize(av, ai):
        m0 = av[:, :1]
        ex = jnp.exp(av - m0)
        w_out_ref[...] = ex / jnp.sum(ex, axis=1, keepdims=True)
        i_out_ref[...] = ai

    finalize(acc_v, acc_i)

    bad = jnp.any(vlast >= gk)

    @pl.when(bad)
    def _fallback():
        cur_ref[...] = jnp.dot(h, en, preferred_element_type=jnp.float32)
        iota = jax.lax.broadcasted_iota(jnp.int32, (T, N), 1)

        def b2(j, carry):
            av, ai = carry
            cur = cur_ref[...]
            mm = jnp.max(cur, axis=1, keepdims=True)
            am = jnp.min(jnp.where(cur == mm, iota, N), axis=1, keepdims=True)
            cur_ref[...] = jnp.where(iota == am, -jnp.inf, cur)
            av = jnp.where(outlane == j, mm, av)
            ai = jnp.where(outlane == j, am, ai)
            return av, ai

        av2, ai2 = jax.lax.fori_loop(0, k, b2, (acc_v0, acc_i0))
        finalize(av2, ai2)


def _route_pool(h, embt, k, m):
    TOK = h.shape[0]
    N = embt.shape[1]
    grid = TOK // TOKENS_BLK
    return pl.pallas_call(
        functools.partial(_pool_kernel, k=k, m=m),
        grid=(grid,),
        in_specs=[
            pl.BlockSpec((TOKENS_BLK, D_SPACE), lambda i: (i, 0)),
            pl.BlockSpec((D_SPACE, N), lambda i: (0, 0)),
        ],
        out_specs=[
            pl.BlockSpec((TOKENS_BLK, OUT_W), lambda i: (i, 0)),
            pl.BlockSpec((TOKENS_BLK, OUT_W), lambda i: (i, 0)),
        ],
        out_shape=[
            jax.ShapeDtypeStruct((TOK, OUT_W), jnp.float32),
            jax.ShapeDtypeStruct((TOK, OUT_W), jnp.int32),
        ],
        scratch_shapes=[
            pltpu.VMEM((TOKENS_BLK, N), jnp.float32),
            pltpu.VMEM((TOKENS_BLK, m * 128), jnp.float32),
            pltpu.VMEM((TOKENS_BLK, m * 128), jnp.int32),
        ],
    )(h, embt)


def kernel(x, W_proj, b_proj, neuron_emb, neuron_emb_rel_k):
    B, S, D = x.shape
    TOK = B * S
    xf = x.reshape(TOK, D)
    grid = TOK // TOKENS_BLK
    h = pl.pallas_call(
        _h_kernel,
        grid=(grid,),
        in_specs=[
            pl.BlockSpec((TOKENS_BLK, D), lambda i: (i, 0)),
            pl.BlockSpec((D, D_SPACE), lambda i: (0, 0)),
            pl.BlockSpec((1, D_SPACE), lambda i: (0, 0)),
        ],
        out_specs=pl.BlockSpec((TOKENS_BLK, D_SPACE), lambda i: (i, 0)),
        out_shape=jax.ShapeDtypeStruct((TOK, D_SPACE), jnp.float32),
    )(xf, W_proj, b_proj.reshape(1, D_SPACE))

    pools = [
        (neuron_emb[0:2048].T, 64, 8),
        (neuron_emb[2048:4096].T, 32, 6),
        (neuron_emb[4096:8192].T, 64, 8),
        (neuron_emb_rel_k.T, 64, 8),
        (neuron_emb[8192:12288].T, 32, 6),
    ]
    ws, idxs = [], []
    for embt, k, m in pools:
        w, i = _route_pool(h, embt, k, m)
        ws.append(w[:, :k])
        idxs.append(i[:, :k])
    weights = jnp.concatenate(ws, axis=1).reshape(B, S, -1)
    indices = jnp.concatenate(idxs, axis=1).reshape(B, S, -1)
    return weights, indices


# speedup vs baseline: 1.1508x; 1.1508x over previous
import functools

import jax
import jax.numpy as jnp
from jax.experimental import pallas as pl
from jax.experimental.pallas import tpu as pltpu

D_MODEL = 1024
D_SPACE = 64
TOKENS_BLK = 256
OUT_W = 128


def _h_kernel(x_ref, w_ref, b_ref, h_ref):
    h_ref[...] = (
        jnp.dot(x_ref[...], w_ref[...], preferred_element_type=jnp.float32)
        + b_ref[...]
    )


def _pool_kernel(h_ref, embt_ref, w_out_ref, i_out_ref, cur_ref, cv_ref,
                 ci_ref, *, k: int, m: int):
    e = embt_ref[...]
    nrm = jnp.sqrt(jnp.sum(e * e, axis=0, keepdims=True))
    en = e / jnp.maximum(nrm, 1e-12)
    h = h_ref[...]
    logits = jnp.dot(h, en, preferred_element_type=jnp.float32)
    cur_ref[...] = logits
    T, N = logits.shape
    C = N // 128
    lane = jax.lax.broadcasted_iota(jnp.int32, (T, 128), 1)
    outlane = jax.lax.broadcasted_iota(jnp.int32, (T, OUT_W), 1)
    acc_v0 = jnp.full((T, OUT_W), -jnp.inf, dtype=jnp.float32)
    acc_i0 = jnp.zeros((T, OUT_W), dtype=jnp.int32)

    for it in range(m):
        mx = cur_ref[:, 0:128]
        ci = jnp.zeros((T, 128), jnp.int32)
        for c in range(1, C):
            s = cur_ref[:, c * 128:(c + 1) * 128]
            gt = s > mx
            mx = jnp.where(gt, s, mx)
            ci = jnp.where(gt, c, ci)
        cv_ref[:, it * 128:(it + 1) * 128] = mx
        ci_ref[:, it * 128:(it + 1) * 128] = ci * 128 + lane
        if it < m - 1:
            for c in range(C):
                s = cur_ref[:, c * 128:(c + 1) * 128]
                cur_ref[:, c * 128:(c + 1) * 128] = jnp.where(
                    (ci == c) & (s == mx), -jnp.inf, s)

    vlast = cv_ref[:, (m - 1) * 128:m * 128]

    BIG = jnp.int32(1 << 30)

    def body(j, carry):
        acc_v, acc_i, hv, hoi, depth, _ = carry
        g = jnp.max(hv, axis=1, keepdims=True)
        eq = hv == g
        win = jnp.min(jnp.where(eq, hoi, BIG), axis=1, keepdims=True)
        winm = eq & (hoi == win)
        nd = depth + winm.astype(jnp.int32)
        t0 = (nd & 1) != 0
        t1 = (nd & 2) != 0
        t2 = (nd & 4) != 0

        def tree(ref):
            cs = [ref[:, c * 128:(c + 1) * 128] for c in range(m)]
            cs = cs + [cs[0]] * (8 - m)
            a = jnp.where(t0, cs[1], cs[0])
            b = jnp.where(t0, cs[3], cs[2])
            c_ = jnp.where(t0, cs[5], cs[4])
            d = jnp.where(t0, cs[7], cs[6])
            e_ = jnp.where(t1, b, a)
            f = jnp.where(t1, d, c_)
            return jnp.where(t2, f, e_)

        newv = jnp.where(nd >= m, -jnp.inf, tree(cv_ref))
        newi = tree(ci_ref)
        hv = jnp.where(winm, newv, hv)
        hoi = jnp.where(winm, newi, hoi)
        sel = outlane == j
        acc_v = jnp.where(sel, g, acc_v)
        acc_i = jnp.where(sel, win, acc_i)
        return acc_v, acc_i, hv, hoi, nd, g

    g0 = jnp.zeros((T, 1), jnp.float32)
    hv0 = cv_ref[:, 0:128]
    hoi0 = ci_ref[:, 0:128]
    d0 = jnp.zeros((T, 128), jnp.int32)
    acc_v, acc_i, _, _, _, gk = jax.lax.fori_loop(
        0, k, body, (acc_v0, acc_i0, hv0, hoi0, d0, g0))

    def finalize(av, ai):
        m0 = av[:, :1]
        ex = jnp.exp(av - m0)
        w_out_ref[...] = ex / jnp.sum(ex, axis=1, keepdims=True)
        i_out_ref[...] = ai

    finalize(acc_v, acc_i)

    bad = jnp.any(vlast >= gk)

    @pl.when(bad)
    def _fallback():
        cur_ref[...] = jnp.dot(h, en, preferred_element_type=jnp.float32)
        iota = jax.lax.broadcasted_iota(jnp.int32, (T, N), 1)

        def b2(j, carry):
            av, ai = carry
            cur = cur_ref[...]
            mm = jnp.max(cur, axis=1, keepdims=True)
            am = jnp.min(jnp.where(cur == mm, iota, N), axis=1, keepdims=True)
            cur_ref[...] = jnp.where(iota == am, -jnp.inf, cur)
            av = jnp.where(outlane == j, mm, av)
            ai = jnp.where(outlane == j, am, ai)
            return av, ai

        av2, ai2 = jax.lax.fori_loop(0, k, b2, (acc_v0, acc_i0))
        finalize(av2, ai2)


def _route_pool(h, embt, k, m):
    TOK = h.shape[0]
    N = embt.shape[1]
    grid = TOK // TOKENS_BLK
    return pl.pallas_call(
        functools.partial(_pool_kernel, k=k, m=m),
        grid=(grid,),
        in_specs=[
            pl.BlockSpec((TOKENS_BLK, D_SPACE), lambda i: (i, 0)),
            pl.BlockSpec((D_SPACE, N), lambda i: (0, 0)),
        ],
        out_specs=[
            pl.BlockSpec((TOKENS_BLK, OUT_W), lambda i: (i, 0)),
            pl.BlockSpec((TOKENS_BLK, OUT_W), lambda i: (i, 0)),
        ],
        out_shape=[
            jax.ShapeDtypeStruct((TOK, OUT_W), jnp.float32),
            jax.ShapeDtypeStruct((TOK, OUT_W), jnp.int32),
        ],
        scratch_shapes=[
            pltpu.VMEM((TOKENS_BLK, N), jnp.float32),
            pltpu.VMEM((TOKENS_BLK, m * 128), jnp.float32),
            pltpu.VMEM((TOKENS_BLK, m * 128), jnp.int32),
        ],
    )(h, embt)


def kernel(x, W_proj, b_proj, neuron_emb, neuron_emb_rel_k):
    B, S, D = x.shape
    TOK = B * S
    xf = x.reshape(TOK, D)
    grid = TOK // TOKENS_BLK
    h = pl.pallas_call(
        _h_kernel,
        grid=(grid,),
        in_specs=[
            pl.BlockSpec((TOKENS_BLK, D), lambda i: (i, 0)),
            pl.BlockSpec((D, D_SPACE), lambda i: (0, 0)),
            pl.BlockSpec((1, D_SPACE), lambda i: (0, 0)),
        ],
        out_specs=pl.BlockSpec((TOKENS_BLK, D_SPACE), lambda i: (i, 0)),
        out_shape=jax.ShapeDtypeStruct((TOK, D_SPACE), jnp.float32),
    )(xf, W_proj, b_proj.reshape(1, D_SPACE))

    pools = [
        (neuron_emb[0:2048].T, 64, 8),
        (neuron_emb[2048:4096].T, 32, 6),
        (neuron_emb[4096:8192].T, 64, 8),
        (neuron_emb_rel_k.T, 64, 8),
        (neuron_emb[8192:12288].T, 32, 6),
    ]
    ws, idxs = [], []
    for embt, k, m in pools:
        w, i = _route_pool(h, embt, k, m)
        ws.append(w[:, :k])
        idxs.append(i[:, :k])
    weights = jnp.concatenate(ws, axis=1).reshape(B, S, -1)
    indices = jnp.concatenate(idxs, axis=1).reshape(B, S, -1)
    return weights, indices


# speedup vs baseline: 1.1752x; 1.0212x over previous
import functools

import jax
import jax.numpy as jnp
from jax.experimental import pallas as pl
from jax.experimental.pallas import tpu as pltpu

D_MODEL = 1024
D_SPACE = 64
TOKENS_BLK = 256
OUT_W = 128


def _h_kernel(x_ref, w_ref, b_ref, h_ref):
    h_ref[...] = (
        jnp.dot(x_ref[...], w_ref[...], preferred_element_type=jnp.float32)
        + b_ref[...]
    )


def _norm_kernel(embt_ref, out_ref):
    e = embt_ref[...]
    nrm = jnp.sqrt(jnp.sum(e * e, axis=0, keepdims=True))
    out_ref[...] = e / jnp.maximum(nrm, 1e-12)


def _normalize_embt(embt):
    return pl.pallas_call(
        _norm_kernel,
        out_shape=jax.ShapeDtypeStruct(embt.shape, jnp.float32),
    )(embt)


def _pool_kernel(h_ref, embt_ref, w_out_ref, i_out_ref, cur_ref, cv_ref,
                 ci_ref, *, k: int, m: int):
    en = embt_ref[...]
    h = h_ref[...]
    logits = jnp.dot(h, en, preferred_element_type=jnp.float32)
    cur_ref[...] = logits
    T, N = logits.shape
    C = N // 128
    lane = jax.lax.broadcasted_iota(jnp.int32, (T, 128), 1)
    outlane = jax.lax.broadcasted_iota(jnp.int32, (T, OUT_W), 1)
    acc_v0 = jnp.full((T, OUT_W), -jnp.inf, dtype=jnp.float32)
    acc_i0 = jnp.zeros((T, OUT_W), dtype=jnp.int32)

    for it in range(m):
        mx = cur_ref[:, 0:128]
        ci = jnp.zeros((T, 128), jnp.int32)
        for c in range(1, C):
            s = cur_ref[:, c * 128:(c + 1) * 128]
            gt = s > mx
            mx = jnp.where(gt, s, mx)
            ci = jnp.where(gt, c, ci)
        cv_ref[:, it * 128:(it + 1) * 128] = mx
        ci_ref[:, it * 128:(it + 1) * 128] = ci * 128 + lane
        if it < m - 1:
            for c in range(C):
                s = cur_ref[:, c * 128:(c + 1) * 128]
                cur_ref[:, c * 128:(c + 1) * 128] = jnp.where(
                    (ci == c) & (s == mx), -jnp.inf, s)

    vlast = cv_ref[:, (m - 1) * 128:m * 128]

    BIG = jnp.int32(1 << 30)

    def body(j, carry):
        acc_v, acc_i, hv, hoi, depth, _ = carry
        g = jnp.max(hv, axis=1, keepdims=True)
        eq = hv == g
        win = jnp.min(jnp.where(eq, hoi, BIG), axis=1, keepdims=True)
        winm = eq & (hoi == win)
        nd = depth + winm.astype(jnp.int32)
        t0 = (nd & 1) != 0
        t1 = (nd & 2) != 0
        t2 = (nd & 4) != 0

        def tree(ref):
            cs = [ref[:, c * 128:(c + 1) * 128] for c in range(m)]
            cs = cs + [cs[0]] * (8 - m)
            a = jnp.where(t0, cs[1], cs[0])
            b = jnp.where(t0, cs[3], cs[2])
            c_ = jnp.where(t0, cs[5], cs[4])
            d = jnp.where(t0, cs[7], cs[6])
            e_ = jnp.where(t1, b, a)
            f = jnp.where(t1, d, c_)
            return jnp.where(t2, f, e_)

        newv = jnp.where(nd >= m, -jnp.inf, tree(cv_ref))
        newi = tree(ci_ref)
        hv = jnp.where(winm, newv, hv)
        hoi = jnp.where(winm, newi, hoi)
        sel = outlane == j
        acc_v = jnp.where(sel, g, acc_v)
        acc_i = jnp.where(sel, win, acc_i)
        return acc_v, acc_i, hv, hoi, nd, g

    g0 = jnp.zeros((T, 1), jnp.float32)
    hv0 = cv_ref[:, 0:128]
    hoi0 = ci_ref[:, 0:128]
    d0 = jnp.zeros((T, 128), jnp.int32)
    acc_v, acc_i, _, _, _, gk = jax.lax.fori_loop(
        0, k, body, (acc_v0, acc_i0, hv0, hoi0, d0, g0))

    def finalize(av, ai):
        m0 = av[:, :1]
        ex = jnp.exp(av - m0)
        w_out_ref[...] = ex / jnp.sum(ex, axis=1, keepdims=True)
        i_out_ref[...] = ai

    finalize(acc_v, acc_i)

    bad = jnp.any(vlast >= gk)

    @pl.when(bad)
    def _fallback():
        cur_ref[...] = jnp.dot(h, en, preferred_element_type=jnp.float32)
        iota = jax.lax.broadcasted_iota(jnp.int32, (T, N), 1)

        def b2(j, carry):
            av, ai = carry
            cur = cur_ref[...]
            mm = jnp.max(cur, axis=1, keepdims=True)
            am = jnp.min(jnp.where(cur == mm, iota, N), axis=1, keepdims=True)
            cur_ref[...] = jnp.where(iota == am, -jnp.inf, cur)
            av = jnp.where(outlane == j, mm, av)
            ai = jnp.where(outlane == j, am, ai)
            return av, ai

        av2, ai2 = jax.lax.fori_loop(0, k, b2, (acc_v0, acc_i0))
        finalize(av2, ai2)


def _route_pool(h, embt, k, m):
    TOK = h.shape[0]
    N = embt.shape[1]
    grid = TOK // TOKENS_BLK
    return pl.pallas_call(
        functools.partial(_pool_kernel, k=k, m=m),
        grid=(grid,),
        in_specs=[
            pl.BlockSpec((TOKENS_BLK, D_SPACE), lambda i: (i, 0)),
            pl.BlockSpec((D_SPACE, N), lambda i: (0, 0)),
        ],
        out_specs=[
            pl.BlockSpec((TOKENS_BLK, OUT_W), lambda i: (i, 0)),
            pl.BlockSpec((TOKENS_BLK, OUT_W), lambda i: (i, 0)),
        ],
        out_shape=[
            jax.ShapeDtypeStruct((TOK, OUT_W), jnp.float32),
            jax.ShapeDtypeStruct((TOK, OUT_W), jnp.int32),
        ],
        scratch_shapes=[
            pltpu.VMEM((TOKENS_BLK, N), jnp.float32),
            pltpu.VMEM((TOKENS_BLK, m * 128), jnp.float32),
            pltpu.VMEM((TOKENS_BLK, m * 128), jnp.int32),
        ],
    )(h, embt)


def kernel(x, W_proj, b_proj, neuron_emb, neuron_emb_rel_k):
    B, S, D = x.shape
    TOK = B * S
    xf = x.reshape(TOK, D)
    grid = TOK // TOKENS_BLK
    h = pl.pallas_call(
        _h_kernel,
        grid=(grid,),
        in_specs=[
            pl.BlockSpec((TOKENS_BLK, D), lambda i: (i, 0)),
            pl.BlockSpec((D, D_SPACE), lambda i: (0, 0)),
            pl.BlockSpec((1, D_SPACE), lambda i: (0, 0)),
        ],
        out_specs=pl.BlockSpec((TOKENS_BLK, D_SPACE), lambda i: (i, 0)),
        out_shape=jax.ShapeDtypeStruct((TOK, D_SPACE), jnp.float32),
    )(xf, W_proj, b_proj.reshape(1, D_SPACE))

    pools = [
        (neuron_emb[0:2048].T, 64, 7),
        (neuron_emb[2048:4096].T, 32, 6),
        (neuron_emb[4096:8192].T, 64, 7),
        (neuron_emb_rel_k.T, 64, 7),
        (neuron_emb[8192:12288].T, 32, 6),
    ]
    ws, idxs = [], []
    for embt, k, m in pools:
        w, i = _route_pool(h, _normalize_embt(embt), k, m)
        ws.append(w[:, :k])
        idxs.append(i[:, :k])
    weights = jnp.concatenate(ws, axis=1).reshape(B, S, -1)
    indices = jnp.concatenate(idxs, axis=1).reshape(B, S, -1)
    return weights, indices
